# Initial kernel scaffold; baseline (speedup 1.0000x reference)
import jax
import jax.numpy as jnp
from jax.experimental import pallas as pl


def kernel(keypoints, bev_features, W, gamma, beta):
    raise NotImplementedError("write your pallas kernel here")


# speedup vs baseline: 1.0024x; 1.0024x over previous
import functools

import jax
import jax.numpy as jnp
import numpy as np
from jax import lax
from jax.experimental import pallas as pl
from jax.experimental.pallas import tpu as pltpu
from jax.experimental.pallas import tpu_sc as plsc

_PC0 = np.float32(-75.2)
_VS = np.float32(0.1)
_STRIDE = np.float32(8.0)

_B = 4
_N = 4096
_H = 188
_W = 188
_HW = _H * _W
_CIN = 256
_COUT = 128
_NPTS = _B * _N
_NROWS = 4 * _NPTS

_PT = 512


def _proj_body(x_ref, w_ref, o_ref):
    x = x_ref[0]
    w = w_ref[...]
    o_ref[0] = lax.dot_general(
        x, w, (((0,), (1,)), ((), ())), preferred_element_type=jnp.float32)


def _project(bev_flat, w):
    return pl.pallas_call(
        _proj_body,
        grid=(_B, (_HW + _PT - 1) // _PT),
        in_specs=[
            pl.BlockSpec((1, _CIN, _PT), lambda b, t: (b, 0, t)),
            pl.BlockSpec((_COUT, _CIN), lambda b, t: (0, 0)),
        ],
        out_specs=pl.BlockSpec((1, _PT, _COUT), lambda b, t: (b, t, 0)),
        out_shape=jax.ShapeDtypeStruct((_B, _HW, _COUT), jnp.float32),
    )(bev_flat, w)


def _idxw_body(xs_ref, ys_ref, idx_ref, w_ref):
    xs = xs_ref[...]
    ys = ys_ref[...]
    xi = ((xs - _PC0) / _VS) / _STRIDE
    yi = ((ys - _PC0) / _VS) / _STRIDE
    x0 = jnp.floor(xi).astype(jnp.int32)
    y0 = jnp.floor(yi).astype(jnp.int32)
    x0c = jnp.clip(x0, 0, _W - 1)
    x1c = jnp.clip(x0 + 1, 0, _W - 1)
    y0c = jnp.clip(y0, 0, _H - 1)
    y1c = jnp.clip(y0 + 1, 0, _H - 1)
    x0f = x0c.astype(jnp.float32)
    x1f = x1c.astype(jnp.float32)
    y0f = y0c.astype(jnp.float32)
    y1f = y1c.astype(jnp.float32)
    w_ref[0] = (x1f - xi) * (y1f - yi)
    w_ref[1] = (x1f - xi) * (yi - y0f)
    w_ref[2] = (xi - x0f) * (y1f - yi)
    w_ref[3] = (xi - x0f) * (yi - y0f)
    base = lax.broadcasted_iota(jnp.int32, (_B, _N), 0) * _HW
    idx_ref[0] = base + y0c * _W + x0c
    idx_ref[1] = base + y1c * _W + x0c
    idx_ref[2] = base + y0c * _W + x1c
    idx_ref[3] = base + y1c * _W + x1c


def _idx_weights(xs, ys):
    return pl.pallas_call(
        _idxw_body,
        in_specs=[
            pl.BlockSpec((_B, _N), lambda: (0, 0)),
            pl.BlockSpec((_B, _N), lambda: (0, 0)),
        ],
        out_specs=[
            pl.BlockSpec((4, _B, _N), lambda: (0, 0, 0)),
            pl.BlockSpec((4, _B, _N), lambda: (0, 0, 0)),
        ],
        out_shape=[
            jax.ShapeDtypeStruct((4, _B, _N), jnp.int32),
            jax.ShapeDtypeStruct((4, _B, _N), jnp.float32),
        ],
    )(xs, ys)


_NW = 32
_RPW = _NROWS // _NW
_Q = 128
_NCH = _RPW // _Q


def _gather_rows(table, idx_flat):
    mesh = plsc.VectorSubcoreMesh(core_axis_name="c", subcore_axis_name="s")

    @functools.partial(
        pl.kernel,
        mesh=mesh,
        out_type=jax.ShapeDtypeStruct((_NROWS, _COUT), jnp.float32),
        scratch_types=[
            pltpu.VMEM((_Q,), jnp.int32),
            pltpu.VMEM((_Q, _COUT), jnp.float32),
            pltpu.SemaphoreType.DMA,
        ],
    )
    def k(table_hbm, idx_hbm, out_hbm, idx_v, rows_v, sem):
        wid = lax.axis_index("s") * 2 + lax.axis_index("c")
        base = pl.multiple_of(wid * _RPW, _Q)
        for c in range(_NCH):
            off = pl.multiple_of(base + c * _Q, _Q)
            pltpu.sync_copy(idx_hbm.at[pl.ds(off, _Q)], idx_v)
            pltpu.async_copy(table_hbm.at[idx_v], rows_v, sem).wait()
            pltpu.sync_copy(rows_v, out_hbm.at[pl.ds(off, _Q)])

    return k(table, idx_flat)


_TP = 1024


def _wsum_body(g_ref, w_ref, h_ref, stats_ref, acc):
    t = pl.program_id(0)
    w = w_ref[...]
    h = (g_ref[0] * w[0][:, None]
         + g_ref[1] * w[1][:, None]
         + g_ref[2] * w[2][:, None]
         + g_ref[3] * w[3][:, None])
    h_ref[...] = h

    @pl.when(t == 0)
    def _():
        acc[...] = jnp.zeros_like(acc)

    acc[0:1] += jnp.sum(h, axis=0, keepdims=True)
    acc[1:2] += jnp.sum(h * h, axis=0, keepdims=True)

    @pl.when(t == pl.num_programs(0) - 1)
    def _():
        stats_ref[...] = acc[...]


def _wsum_stats(g, w4):
    return pl.pallas_call(
        _wsum_body,
        grid=(_NPTS // _TP,),
        in_specs=[
            pl.BlockSpec((4, _TP, _COUT), lambda t: (0, t, 0)),
            pl.BlockSpec((4, _TP), lambda t: (0, t)),
        ],
        out_specs=[
            pl.BlockSpec((_TP, _COUT), lambda t: (t, 0)),
            pl.BlockSpec((8, _COUT), lambda t: (0, 0)),
        ],
        out_shape=[
            jax.ShapeDtypeStruct((_NPTS, _COUT), jnp.float32),
            jax.ShapeDtypeStruct((8, _COUT), jnp.float32),
        ],
        scratch_shapes=[pltpu.VMEM((8, _COUT), jnp.float32)],
    )(g, w4)


_TB = 2048


def _bn_body(h_ref, stats_ref, gam_ref, bet_ref, o_ref):
    inv_n = np.float32(1.0 / _NPTS)
    mean = stats_ref[0:1] * inv_n
    var = stats_ref[1:2] * inv_n - mean * mean
    hn = (h_ref[...] - mean) / jnp.sqrt(var + np.float32(1e-5))
    o_ref[...] = jnp.maximum(hn * gam_ref[...] + bet_ref[...], 0.0)


def _bn_relu(h, stats, gamma2d, beta2d):
    return pl.pallas_call(
        _bn_body,
        grid=(_NPTS // _TB,),
        in_specs=[
            pl.BlockSpec((_TB, _COUT), lambda t: (t, 0)),
            pl.BlockSpec((8, _COUT), lambda t: (0, 0)),
            pl.BlockSpec((1, _COUT), lambda t: (0, 0)),
            pl.BlockSpec((1, _COUT), lambda t: (0, 0)),
        ],
        out_specs=pl.BlockSpec((_TB, _COUT), lambda t: (t, 0)),
        out_shape=jax.ShapeDtypeStruct((_NPTS, _COUT), jnp.float32),
    )(h, stats, gamma2d, beta2d)


def kernel(keypoints, bev_features, W, gamma, beta):
    bev_flat = bev_features.reshape(_B, _CIN, _HW)
    proj = _project(bev_flat, W).reshape(_B * _HW, _COUT)

    xs = keypoints[:, :, 0]
    ys = keypoints[:, :, 1]
    idx4, w4 = _idx_weights(xs, ys)

    g = _gather_rows(proj, idx4.reshape(_NROWS)).reshape(4, _NPTS, _COUT)
    h, stats = _wsum_stats(g, w4.reshape(4, _NPTS))
    return _bn_relu(h, stats, gamma.reshape(1, _COUT), beta.reshape(1, _COUT))


# speedup vs baseline: 1.3413x; 1.3381x over previous
import functools

import jax
import jax.numpy as jnp
import numpy as np
from jax import lax
from jax.experimental import pallas as pl
from jax.experimental.pallas import tpu as pltpu
from jax.experimental.pallas import tpu_sc as plsc

_PC0 = np.float32(-75.2)
_VS = np.float32(0.1)
_STRIDE = np.float32(8.0)

_B = 4
_N = 4096
_H = 188
_W = 188
_HW = _H * _W
_CIN = 256
_COUT = 128
_NPTS = _B * _N
_NROWS = 4 * _NPTS

_PT = 2048


def _proj_body(x_ref, w_ref, o_ref):
    x = x_ref[0]
    w = w_ref[...]
    o_ref[0] = lax.dot_general(
        x, w, (((0,), (1,)), ((), ())), preferred_element_type=jnp.float32)


def _project(bev_flat, w):
    return pl.pallas_call(
        _proj_body,
        grid=(_B, (_HW + _PT - 1) // _PT),
        in_specs=[
            pl.BlockSpec((1, _CIN, _PT), lambda b, t: (b, 0, t)),
            pl.BlockSpec((_COUT, _CIN), lambda b, t: (0, 0)),
        ],
        out_specs=pl.BlockSpec((1, _PT, _COUT), lambda b, t: (b, t, 0)),
        out_shape=jax.ShapeDtypeStruct((_B, _HW, _COUT), jnp.float32),
        compiler_params=pltpu.CompilerParams(
            dimension_semantics=("parallel", "parallel")),
    )(bev_flat, w)


def _idxw_body(xs_ref, ys_ref, idx_ref, w_ref):
    xs = xs_ref[...]
    ys = ys_ref[...]
    xi = ((xs - _PC0) / _VS) / _STRIDE
    yi = ((ys - _PC0) / _VS) / _STRIDE
    x0 = jnp.floor(xi).astype(jnp.int32)
    y0 = jnp.floor(yi).astype(jnp.int32)
    x0c = jnp.clip(x0, 0, _W - 1)
    x1c = jnp.clip(x0 + 1, 0, _W - 1)
    y0c = jnp.clip(y0, 0, _H - 1)
    y1c = jnp.clip(y0 + 1, 0, _H - 1)
    x0f = x0c.astype(jnp.float32)
    x1f = x1c.astype(jnp.float32)
    y0f = y0c.astype(jnp.float32)
    y1f = y1c.astype(jnp.float32)
    w_ref[0] = (x1f - xi) * (y1f - yi)
    w_ref[1] = (x1f - xi) * (yi - y0f)
    w_ref[2] = (xi - x0f) * (y1f - yi)
    w_ref[3] = (xi - x0f) * (yi - y0f)
    base = lax.broadcasted_iota(jnp.int32, (_B, _N), 0) * _HW
    idx_ref[0] = base + y0c * _W + x0c
    idx_ref[1] = base + y1c * _W + x0c
    idx_ref[2] = base + y0c * _W + x1c
    idx_ref[3] = base + y1c * _W + x1c


def _idx_weights(xs, ys):
    return pl.pallas_call(
        _idxw_body,
        in_specs=[
            pl.BlockSpec((_B, _N), lambda: (0, 0)),
            pl.BlockSpec((_B, _N), lambda: (0, 0)),
        ],
        out_specs=[
            pl.BlockSpec((4, _B, _N), lambda: (0, 0, 0)),
            pl.BlockSpec((4, _B, _N), lambda: (0, 0, 0)),
        ],
        out_shape=[
            jax.ShapeDtypeStruct((4, _B, _N), jnp.int32),
            jax.ShapeDtypeStruct((4, _B, _N), jnp.float32),
        ],
    )(xs, ys)


_NW = 32
_RPW = _NROWS // _NW
_Q = 128
_NCH = _RPW // _Q


def _gather_rows(table, idx_flat):
    mesh = plsc.VectorSubcoreMesh(core_axis_name="c", subcore_axis_name="s")

    @functools.partial(
        pl.kernel,
        mesh=mesh,
        out_type=jax.ShapeDtypeStruct((_NROWS, _COUT), jnp.float32),
        scratch_types=[
            pltpu.VMEM((_Q,), jnp.int32),
            pltpu.VMEM((_Q, _COUT), jnp.float32),
            pltpu.SemaphoreType.DMA,
        ],
    )
    def k(table_hbm, idx_hbm, out_hbm, idx_v, rows_v, sem):
        wid = lax.axis_index("s") * 2 + lax.axis_index("c")
        base = pl.multiple_of(wid * _RPW, _Q)
        for c in range(_NCH):
            off = pl.multiple_of(base + c * _Q, _Q)
            pltpu.sync_copy(idx_hbm.at[pl.ds(off, _Q)], idx_v)
            pltpu.async_copy(table_hbm.at[idx_v], rows_v, sem).wait()
            pltpu.sync_copy(rows_v, out_hbm.at[pl.ds(off, _Q)])

    return k(table, idx_flat)


_TP = 1024


def _wsum_body(g_ref, w_ref, h_ref, stats_ref, acc):
    t = pl.program_id(0)
    w = w_ref[...]
    g = g_ref[...]
    h = (g[0] * w[0][:, None]
         + g[1] * w[1][:, None]
         + g[2] * w[2][:, None]
         + g[3] * w[3][:, None])
    h_ref[...] = h

    @pl.when(t == 0)
    def _():
        acc[...] = jnp.zeros_like(acc)

    acc[0:1] += jnp.sum(h, axis=0, keepdims=True)
    acc[1:2] += jnp.sum(h * h, axis=0, keepdims=True)

    @pl.when(t == pl.num_programs(0) - 1)
    def _():
        stats_ref[...] = acc[...]


def _wsum_stats(g, w4):
    return pl.pallas_call(
        _wsum_body,
        grid=(_NPTS // _TP,),
        in_specs=[
            pl.BlockSpec((4, _TP, _COUT), lambda t: (0, t, 0)),
            pl.BlockSpec((4, _TP), lambda t: (0, t)),
        ],
        out_specs=[
            pl.BlockSpec((_TP, _COUT), lambda t: (t, 0)),
            pl.BlockSpec((8, _COUT), lambda t: (0, 0)),
        ],
        out_shape=[
            jax.ShapeDtypeStruct((_NPTS, _COUT), jnp.float32),
            jax.ShapeDtypeStruct((8, _COUT), jnp.float32),
        ],
        scratch_shapes=[pltpu.VMEM((8, _COUT), jnp.float32)],
    )(g, w4)


_TB = 2048


def _bn_body(h_ref, stats_ref, gam_ref, bet_ref, o_ref):
    inv_n = np.float32(1.0 / _NPTS)
    mean = stats_ref[0:1] * inv_n
    var = stats_ref[1:2] * inv_n - mean * mean
    hn = (h_ref[...] - mean) / jnp.sqrt(var + np.float32(1e-5))
    o_ref[...] = jnp.maximum(hn * gam_ref[...] + bet_ref[...], 0.0)


def _bn_relu(h, stats, gamma2d, beta2d):
    return pl.pallas_call(
        _bn_body,
        grid=(_NPTS // _TB,),
        in_specs=[
            pl.BlockSpec((_TB, _COUT), lambda t: (t, 0)),
            pl.BlockSpec((8, _COUT), lambda t: (0, 0)),
            pl.BlockSpec((1, _COUT), lambda t: (0, 0)),
            pl.BlockSpec((1, _COUT), lambda t: (0, 0)),
        ],
        out_specs=pl.BlockSpec((_TB, _COUT), lambda t: (t, 0)),
        out_shape=jax.ShapeDtypeStruct((_NPTS, _COUT), jnp.float32),
    )(h, stats, gamma2d, beta2d)


def kernel(keypoints, bev_features, W, gamma, beta):
    bev_flat = bev_features.reshape(_B, _CIN, _HW)
    proj = _project(bev_flat, W).reshape(_B * _HW, _COUT)

    xs = keypoints[:, :, 0]
    ys = keypoints[:, :, 1]
    idx4, w4 = _idx_weights(xs, ys)

    g = _gather_rows(proj, idx4.reshape(_NROWS)).reshape(4, _NPTS, _COUT)
    h, stats = _wsum_stats(g, w4.reshape(4, _NPTS))
    return _bn_relu(h, stats, gamma.reshape(1, _COUT), beta.reshape(1, _COUT))
